# baseline (device time: 51214 ns/iter reference)
import jax
import jax.numpy as jnp
from jax import lax
from jax.experimental import pallas as pl
from jax.experimental.pallas import tpu as pltpu

N_DEV = 8
SQ = 1024
D = 1024
HQ_LOC = 8
DH = 128
BLK = 64
RC = 256
CH = 64
SCALE = 0.08838834764831843


def kernel(x, Wq, K_ext, V_ext, Wo):
    def body(x_ref, wq_ref, k_hbm, v_hbm, wo_ref, out_ref,
             kland, vland, accb, recvA, recvB, recvC, recvD, recvE,
             copy_sems, send_sems, recv_sems):
        my_pos = lax.axis_index("i")

        hsl = pl.ds(my_pos * HQ_LOC, HQ_LOC)
        kcopy = pltpu.make_async_copy(k_hbm.at[0, :, hsl, :], kland,
                                      copy_sems.at[0])
        vcopy = pltpu.make_async_copy(v_hbm.at[0, :, hsl, :], vland,
                                      copy_sems.at[1])
        kcopy.start()
        vcopy.start()

        barrier_sem = pltpu.get_barrier_semaphore()
        for j in range(N_DEV - 1):
            pl.semaphore_signal(barrier_sem, inc=1,
                                device_id=(lax.rem(my_pos + j + 1, N_DEV),),
                                device_id_type=pl.DeviceIdType.MESH)

        wqb = (wq_ref[...] * SCALE).astype(jnp.bfloat16)
        wob = wo_ref[...].astype(jnp.bfloat16)
        pl.semaphore_wait(barrier_sem, N_DEV - 1)
        kcopy.wait()
        vcopy.wait()
        kb = kland[...].astype(jnp.bfloat16)
        vb = vland[...].astype(jnp.bfloat16)

        def mk_dmask(H):
            row_blk = lax.broadcasted_iota(jnp.int32, (H, H), 0) // BLK
            col_blk = lax.broadcasted_iota(jnp.int32, (H, H), 1) // BLK
            return (col_blk <= row_blk).astype(jnp.bfloat16)

        dm256 = mk_dmask(256)
        dm128 = mk_dmask(128)

        def compute_chunk(r0, H, dm):
            ext = r0 + H
            rows = slice(r0, r0 + H)
            xc = x_ref[0, rows, :].astype(jnp.bfloat16)
            qc = jnp.dot(xc, wqb,
                         preferred_element_type=jnp.float32).astype(jnp.bfloat16)
            ctxs = []
            for h in range(HQ_LOC):
                q_h = qc[:, h * DH:(h + 1) * DH]
                s = lax.dot_general(q_h, kb[:ext, h, :],
                                    (((1,), (1,)), ((), ())),
                                    preferred_element_type=jnp.float32)
                w = jnp.exp(s.astype(jnp.bfloat16))
                wd = w[:, r0:] * dm
                wsum = jnp.sum(wd, axis=1, keepdims=True,
                               dtype=jnp.float32)
                ctx = jnp.dot(wd, vb[r0:ext, h, :],
                              preferred_element_type=jnp.float32)
                if r0 > 0:
                    wv = w[:, :r0]
                    wsum = wsum + jnp.sum(wv, axis=1, keepdims=True,
                                          dtype=jnp.float32)
                    ctx = ctx + jnp.dot(wv, vb[:r0, h, :],
                                        preferred_element_type=jnp.float32)
                ctxs.append((ctx / wsum).astype(jnp.bfloat16))
            ctx_c = jnp.concatenate(ctxs, axis=1)
            accb[rows, :] = jnp.dot(
                ctx_c, wob, preferred_element_type=jnp.float32
            ).astype(jnp.bfloat16)

        def peer(j):
            return lax.rem(my_pos + j + 1, N_DEV)

        def rs_issue(base, ch, sem0, rbuf):
            rds = []
            for j in range(N_DEV - 1):
                p = peer(j)
                rd = pltpu.make_async_remote_copy(
                    src_ref=accb.at[pl.ds(base + p * ch, ch), :],
                    dst_ref=rbuf.at[6 - j],
                    send_sem=send_sems.at[sem0 + j],
                    recv_sem=recv_sems.at[sem0 + 6 - j],
                    device_id=(p,),
                    device_id_type=pl.DeviceIdType.MESH,
                )
                rd.start()
                rds.append(rd)
            return rds

        def rs_reduce(base, ch, rbuf):
            sl = pl.ds(base + my_pos * ch, ch)
            s = accb[sl, :].astype(jnp.float32)
            for k in range(N_DEV - 1):
                s = s + rbuf[k].astype(jnp.float32)
            accb[sl, :] = s.astype(jnp.bfloat16)

        def ag_issue(base, ch, sem0):
            sl = pl.ds(base + my_pos * ch, ch)
            rds = []
            for j in range(N_DEV - 1):
                rd = pltpu.make_async_remote_copy(
                    src_ref=accb.at[sl, :],
                    dst_ref=accb.at[sl, :],
                    send_sem=send_sems.at[sem0 + j],
                    recv_sem=recv_sems.at[sem0 + 6 - j],
                    device_id=(peer(j),),
                    device_id_type=pl.DeviceIdType.MESH,
                )
                rd.start()
                rds.append(rd)
            return rds

        def wait_all(rds):
            for rd in rds:
                rd.wait()

        compute_chunk(768, 256, dm256)
        gA = rs_issue(768, 32, 0, recvA)
        compute_chunk(512, 256, dm256)
        gB = rs_issue(512, 32, 14, recvB)
        compute_chunk(256, 256, dm256)
        gC = rs_issue(256, 32, 28, recvC)
        wait_all(gA)
        rs_reduce(768, 32, recvA)
        agA = ag_issue(768, 32, 7)
        compute_chunk(128, 128, dm128)
        gD = rs_issue(128, 16, 42, recvD)
        wait_all(gB)
        rs_reduce(512, 32, recvB)
        agB = ag_issue(512, 32, 21)
        compute_chunk(0, 128, dm128)
        gE = rs_issue(0, 16, 56, recvE)
        wait_all(gC)
        rs_reduce(256, 32, recvC)
        agC = ag_issue(256, 32, 35)
        wait_all(agA)
        out_ref[0, 768:, :] = accb[768:, :].astype(jnp.float32)
        wait_all(gD)
        rs_reduce(128, 16, recvD)
        agD = ag_issue(128, 16, 49)
        wait_all(agB)
        out_ref[0, 512:768, :] = accb[512:768, :].astype(jnp.float32)
        wait_all(gE)
        rs_reduce(0, 16, recvE)
        agE = ag_issue(0, 16, 63)
        wait_all(agC)
        out_ref[0, 256:512, :] = accb[256:512, :].astype(jnp.float32)
        wait_all(agD)
        out_ref[0, 128:256, :] = accb[128:256, :].astype(jnp.float32)
        wait_all(agE)
        out_ref[0, :128, :] = accb[:128, :].astype(jnp.float32)

    return pl.pallas_call(
        body,
        out_shape=jax.ShapeDtypeStruct((1, SQ, D), jnp.float32),
        in_specs=[
            pl.BlockSpec(memory_space=pltpu.VMEM),
            pl.BlockSpec(memory_space=pltpu.VMEM),
            pl.BlockSpec(memory_space=pl.ANY),
            pl.BlockSpec(memory_space=pl.ANY),
            pl.BlockSpec(memory_space=pltpu.VMEM),
        ],
        out_specs=pl.BlockSpec(memory_space=pltpu.VMEM),
        scratch_shapes=[
            pltpu.VMEM((SQ, HQ_LOC, DH), jnp.float32),
            pltpu.VMEM((SQ, HQ_LOC, DH), jnp.float32),
            pltpu.VMEM((SQ, D), jnp.bfloat16),
            pltpu.VMEM((N_DEV - 1, 32, D), jnp.bfloat16),
            pltpu.VMEM((N_DEV - 1, 32, D), jnp.bfloat16),
            pltpu.VMEM((N_DEV - 1, 32, D), jnp.bfloat16),
            pltpu.VMEM((N_DEV - 1, 16, D), jnp.bfloat16),
            pltpu.VMEM((N_DEV - 1, 16, D), jnp.bfloat16),
            pltpu.SemaphoreType.DMA((2,)),
            pltpu.SemaphoreType.DMA((70,)),
            pltpu.SemaphoreType.DMA((70,)),
        ],
        compiler_params=pltpu.CompilerParams(collective_id=0),
    )(x, Wq, K_ext, V_ext, Wo)


# device time: 45626 ns/iter; 1.1225x vs baseline; 1.1225x over previous
import jax
import jax.numpy as jnp
from jax import lax
from jax.experimental import pallas as pl
from jax.experimental.pallas import tpu as pltpu

N_DEV = 8
SQ = 1024
D = 1024
HQ_LOC = 8
DH = 128
BLK = 64
RC = 256
CH = 64
SCALE = 0.08838834764831843


def kernel(x, Wq, K_ext, V_ext, Wo):
    def body(x_ref, wq_ref, k_hbm, v_hbm, wo_ref, out_ref,
             kland, vland, accb, recvA, recvB, recvC, recvD, recvE,
             copy_sems, send_sems, recv_sems):
        my_pos = lax.axis_index("i")

        hsl = pl.ds(my_pos * HQ_LOC, HQ_LOC)
        kcopy = pltpu.make_async_copy(k_hbm.at[0, :, hsl, :], kland,
                                      copy_sems.at[0])
        vcopy = pltpu.make_async_copy(v_hbm.at[0, :, hsl, :], vland,
                                      copy_sems.at[1])
        kcopy.start()
        vcopy.start()

        barrier_sem = pltpu.get_barrier_semaphore()
        for j in range(N_DEV - 1):
            pl.semaphore_signal(barrier_sem, inc=1,
                                device_id=(lax.rem(my_pos + j + 1, N_DEV),),
                                device_id_type=pl.DeviceIdType.MESH)

        wqb = (wq_ref[...] * SCALE).astype(jnp.bfloat16)
        wob = wo_ref[...].astype(jnp.bfloat16)
        pl.semaphore_wait(barrier_sem, N_DEV - 1)
        kcopy.wait()
        vcopy.wait()
        kb = kland[...].astype(jnp.bfloat16)
        vb = vland[...].astype(jnp.bfloat16)

        def mk_dmask(H):
            row_blk = lax.broadcasted_iota(jnp.int32, (H, H), 0) // BLK
            col_blk = lax.broadcasted_iota(jnp.int32, (H, H), 1) // BLK
            return (col_blk <= row_blk).astype(jnp.bfloat16)

        dm256 = mk_dmask(256)
        dm128 = mk_dmask(128)

        def compute_chunk(r0, H, dm):
            ext = r0 + H
            rows = slice(r0, r0 + H)
            xc = x_ref[0, rows, :].astype(jnp.bfloat16)
            qc = jnp.dot(xc, wqb,
                         preferred_element_type=jnp.float32).astype(jnp.bfloat16)
            ctxs = []
            for h in range(HQ_LOC):
                q_h = qc[:, h * DH:(h + 1) * DH]
                s = lax.dot_general(q_h, kb[:ext, h, :],
                                    (((1,), (1,)), ((), ())),
                                    preferred_element_type=jnp.float32)
                w = jnp.exp(s.astype(jnp.bfloat16))
                wd = w[:, r0:] * dm
                wsum = jnp.sum(wd, axis=1, keepdims=True,
                               dtype=jnp.float32)
                ctx = jnp.dot(wd, vb[r0:ext, h, :],
                              preferred_element_type=jnp.float32)
                if r0 > 0:
                    wv = w[:, :r0]
                    wsum = wsum + jnp.sum(wv, axis=1, keepdims=True,
                                          dtype=jnp.float32)
                    ctx = ctx + jnp.dot(wv, vb[:r0, h, :],
                                        preferred_element_type=jnp.float32)
                ctxs.append((ctx / wsum).astype(jnp.bfloat16))
            ctx_c = jnp.concatenate(ctxs, axis=1)
            accb[rows, :] = jnp.dot(
                ctx_c, wob, preferred_element_type=jnp.float32
            ).astype(jnp.bfloat16)

        def peer(j):
            return lax.rem(my_pos + j + 1, N_DEV)

        def rs_issue(base, ch, sem0, rbuf):
            rds = []
            for j in range(N_DEV - 1):
                p = peer(j)
                rd = pltpu.make_async_remote_copy(
                    src_ref=accb.at[pl.ds(base + p * ch, ch), :],
                    dst_ref=rbuf.at[6 - j],
                    send_sem=send_sems.at[sem0 + j],
                    recv_sem=recv_sems.at[sem0 + 6 - j],
                    device_id=(p,),
                    device_id_type=pl.DeviceIdType.MESH,
                )
                rd.start()
                rds.append(rd)
            return rds

        def rs_reduce(base, ch, rbuf):
            sl = pl.ds(base + my_pos * ch, ch)
            s = accb[sl, :].astype(jnp.float32)
            for k in range(N_DEV - 1):
                s = s + rbuf[k].astype(jnp.float32)
            accb[sl, :] = s.astype(jnp.bfloat16)

        def ag_issue(base, ch, sem0):
            sl = pl.ds(base + my_pos * ch, ch)
            rds = []
            for j in range(N_DEV - 1):
                rd = pltpu.make_async_remote_copy(
                    src_ref=accb.at[sl, :],
                    dst_ref=accb.at[sl, :],
                    send_sem=send_sems.at[sem0 + j],
                    recv_sem=recv_sems.at[sem0 + 6 - j],
                    device_id=(peer(j),),
                    device_id_type=pl.DeviceIdType.MESH,
                )
                rd.start()
                rds.append(rd)
            return rds

        def wait_all(rds):
            for rd in rds:
                rd.wait()

        compute_chunk(0, 256, dm256)
        gA = rs_issue(0, 32, 0, recvA)
        compute_chunk(256, 256, dm256)
        gB = rs_issue(256, 32, 14, recvB)
        compute_chunk(512, 256, dm256)
        gC = rs_issue(512, 32, 28, recvC)
        wait_all(gA)
        rs_reduce(0, 32, recvA)
        agA = ag_issue(0, 32, 7)
        wait_all(gB)
        rs_reduce(256, 32, recvB)
        agB = ag_issue(256, 32, 21)
        compute_chunk(768, 128, dm128)
        gD = rs_issue(768, 16, 42, recvD)
        compute_chunk(896, 128, dm128)
        gE = rs_issue(896, 16, 56, recvE)
        wait_all(agA)
        out_ref[0, :256, :] = accb[:256, :].astype(jnp.float32)
        wait_all(gC)
        rs_reduce(512, 32, recvC)
        agC = ag_issue(512, 32, 35)
        wait_all(agB)
        out_ref[0, 256:512, :] = accb[256:512, :].astype(jnp.float32)
        wait_all(gD)
        rs_reduce(768, 16, recvD)
        agD = ag_issue(768, 16, 49)
        wait_all(gE)
        rs_reduce(896, 16, recvE)
        agE = ag_issue(896, 16, 63)
        wait_all(agC)
        out_ref[0, 512:768, :] = accb[512:768, :].astype(jnp.float32)
        wait_all(agD)
        out_ref[0, 768:896, :] = accb[768:896, :].astype(jnp.float32)
        wait_all(agE)
        out_ref[0, 896:, :] = accb[896:, :].astype(jnp.float32)

    return pl.pallas_call(
        body,
        out_shape=jax.ShapeDtypeStruct((1, SQ, D), jnp.float32),
        in_specs=[
            pl.BlockSpec(memory_space=pltpu.VMEM),
            pl.BlockSpec(memory_space=pltpu.VMEM),
            pl.BlockSpec(memory_space=pl.ANY),
            pl.BlockSpec(memory_space=pl.ANY),
            pl.BlockSpec(memory_space=pltpu.VMEM),
        ],
        out_specs=pl.BlockSpec(memory_space=pltpu.VMEM),
        scratch_shapes=[
            pltpu.VMEM((SQ, HQ_LOC, DH), jnp.float32),
            pltpu.VMEM((SQ, HQ_LOC, DH), jnp.float32),
            pltpu.VMEM((SQ, D), jnp.bfloat16),
            pltpu.VMEM((N_DEV - 1, 32, D), jnp.bfloat16),
            pltpu.VMEM((N_DEV - 1, 32, D), jnp.bfloat16),
            pltpu.VMEM((N_DEV - 1, 32, D), jnp.bfloat16),
            pltpu.VMEM((N_DEV - 1, 16, D), jnp.bfloat16),
            pltpu.VMEM((N_DEV - 1, 16, D), jnp.bfloat16),
            pltpu.SemaphoreType.DMA((2,)),
            pltpu.SemaphoreType.DMA((70,)),
            pltpu.SemaphoreType.DMA((70,)),
        ],
        compiler_params=pltpu.CompilerParams(collective_id=0),
    )(x, Wq, K_ext, V_ext, Wo)


# device time: 45121 ns/iter; 1.1350x vs baseline; 1.0112x over previous
import jax
import jax.numpy as jnp
from jax import lax
from jax.experimental import pallas as pl
from jax.experimental.pallas import tpu as pltpu

N_DEV = 8
SQ = 1024
D = 1024
HQ_LOC = 8
DH = 128
BLK = 64
RC = 256
CH = 64
SCALE = 0.08838834764831843


def kernel(x, Wq, K_ext, V_ext, Wo):
    def body(x_ref, wq_ref, k_hbm, v_hbm, wo_ref, out_ref,
             kland, vland, accb, recvA, recvB, recvC, recvD,
             copy_sems, send_sems, recv_sems):
        my_pos = lax.axis_index("i")

        hsl = pl.ds(my_pos * HQ_LOC, HQ_LOC)
        kcopy = pltpu.make_async_copy(k_hbm.at[0, :, hsl, :], kland,
                                      copy_sems.at[0])
        vcopy = pltpu.make_async_copy(v_hbm.at[0, :, hsl, :], vland,
                                      copy_sems.at[1])
        kcopy.start()
        vcopy.start()

        barrier_sem = pltpu.get_barrier_semaphore()
        for j in range(N_DEV - 1):
            pl.semaphore_signal(barrier_sem, inc=1,
                                device_id=(lax.rem(my_pos + j + 1, N_DEV),),
                                device_id_type=pl.DeviceIdType.MESH)

        wqb = (wq_ref[...] * SCALE).astype(jnp.bfloat16)
        wob = wo_ref[...].astype(jnp.bfloat16)
        pl.semaphore_wait(barrier_sem, N_DEV - 1)
        kcopy.wait()
        vcopy.wait()
        kb = kland[...].astype(jnp.bfloat16)
        vb = vland[...].astype(jnp.bfloat16)

        def mk_dmask(H):
            row_blk = lax.broadcasted_iota(jnp.int32, (H, H), 0) // BLK
            col_blk = lax.broadcasted_iota(jnp.int32, (H, H), 1) // BLK
            return (col_blk <= row_blk).astype(jnp.bfloat16)

        dm256 = mk_dmask(256)

        def compute_chunk(r0, H, dm):
            ext = r0 + H
            rows = slice(r0, r0 + H)
            xc = x_ref[0, rows, :].astype(jnp.bfloat16)
            qc = jnp.dot(xc, wqb,
                         preferred_element_type=jnp.float32).astype(jnp.bfloat16)
            ctxs = []
            for h in range(HQ_LOC):
                q_h = qc[:, h * DH:(h + 1) * DH]
                s = lax.dot_general(q_h, kb[:ext, h, :],
                                    (((1,), (1,)), ((), ())),
                                    preferred_element_type=jnp.float32)
                w = jnp.exp(s.astype(jnp.bfloat16))
                wd = w[:, r0:] * dm
                wsum = jnp.sum(wd, axis=1, keepdims=True,
                               dtype=jnp.float32)
                ctx = jnp.dot(wd, vb[r0:ext, h, :],
                              preferred_element_type=jnp.float32)
                if r0 > 0:
                    wv = w[:, :r0]
                    wsum = wsum + jnp.sum(wv, axis=1, keepdims=True,
                                          dtype=jnp.float32)
                    ctx = ctx + jnp.dot(wv, vb[:r0, h, :],
                                        preferred_element_type=jnp.float32)
                ctxs.append((ctx / wsum).astype(jnp.bfloat16))
            ctx_c = jnp.concatenate(ctxs, axis=1)
            accb[rows, :] = jnp.dot(
                ctx_c, wob, preferred_element_type=jnp.float32
            ).astype(jnp.bfloat16)

        def peer(j):
            return lax.rem(my_pos + j + 1, N_DEV)

        def rs_issue(base, ch, sem0, rbuf):
            rds = []
            for j in range(N_DEV - 1):
                p = peer(j)
                rd = pltpu.make_async_remote_copy(
                    src_ref=accb.at[pl.ds(base + p * ch, ch), :],
                    dst_ref=rbuf.at[6 - j],
                    send_sem=send_sems.at[sem0 + j],
                    recv_sem=recv_sems.at[sem0 + 6 - j],
                    device_id=(p,),
                    device_id_type=pl.DeviceIdType.MESH,
                )
                rd.start()
                rds.append(rd)
            return rds

        def rs_reduce(base, ch, rbuf):
            sl = pl.ds(base + my_pos * ch, ch)
            s = accb[sl, :].astype(jnp.float32)
            for k in range(N_DEV - 1):
                s = s + rbuf[k].astype(jnp.float32)
            accb[sl, :] = s.astype(jnp.bfloat16)

        def ag_issue(base, ch, sem0):
            sl = pl.ds(base + my_pos * ch, ch)
            rds = []
            for j in range(N_DEV - 1):
                rd = pltpu.make_async_remote_copy(
                    src_ref=accb.at[sl, :],
                    dst_ref=accb.at[sl, :],
                    send_sem=send_sems.at[sem0 + j],
                    recv_sem=recv_sems.at[sem0 + 6 - j],
                    device_id=(peer(j),),
                    device_id_type=pl.DeviceIdType.MESH,
                )
                rd.start()
                rds.append(rd)
            return rds

        def wait_all(rds):
            for rd in rds:
                rd.wait()

        compute_chunk(0, 256, dm256)
        gA = rs_issue(0, 32, 0, recvA)
        compute_chunk(256, 256, dm256)
        gB = rs_issue(256, 32, 14, recvB)
        compute_chunk(512, 256, dm256)
        gC = rs_issue(512, 32, 28, recvC)
        wait_all(gA)
        rs_reduce(0, 32, recvA)
        agA = ag_issue(0, 32, 7)
        wait_all(gB)
        rs_reduce(256, 32, recvB)
        agB = ag_issue(256, 32, 21)
        compute_chunk(768, 256, dm256)
        gD = rs_issue(768, 32, 42, recvD)
        wait_all(agA)
        out_ref[0, :256, :] = accb[:256, :].astype(jnp.float32)
        wait_all(gC)
        rs_reduce(512, 32, recvC)
        agC = ag_issue(512, 32, 35)
        wait_all(agB)
        out_ref[0, 256:512, :] = accb[256:512, :].astype(jnp.float32)
        wait_all(gD)
        rs_reduce(768, 32, recvD)
        agD = ag_issue(768, 32, 49)
        wait_all(agC)
        out_ref[0, 512:768, :] = accb[512:768, :].astype(jnp.float32)
        wait_all(agD)
        out_ref[0, 768:, :] = accb[768:, :].astype(jnp.float32)

    return pl.pallas_call(
        body,
        out_shape=jax.ShapeDtypeStruct((1, SQ, D), jnp.float32),
        in_specs=[
            pl.BlockSpec(memory_space=pltpu.VMEM),
            pl.BlockSpec(memory_space=pltpu.VMEM),
            pl.BlockSpec(memory_space=pl.ANY),
            pl.BlockSpec(memory_space=pl.ANY),
            pl.BlockSpec(memory_space=pltpu.VMEM),
        ],
        out_specs=pl.BlockSpec(memory_space=pltpu.VMEM),
        scratch_shapes=[
            pltpu.VMEM((SQ, HQ_LOC, DH), jnp.float32),
            pltpu.VMEM((SQ, HQ_LOC, DH), jnp.float32),
            pltpu.VMEM((SQ, D), jnp.bfloat16),
            pltpu.VMEM((N_DEV - 1, 32, D), jnp.bfloat16),
            pltpu.VMEM((N_DEV - 1, 32, D), jnp.bfloat16),
            pltpu.VMEM((N_DEV - 1, 32, D), jnp.bfloat16),
            pltpu.VMEM((N_DEV - 1, 32, D), jnp.bfloat16),
            pltpu.SemaphoreType.DMA((2,)),
            pltpu.SemaphoreType.DMA((56,)),
            pltpu.SemaphoreType.DMA((56,)),
        ],
        compiler_params=pltpu.CompilerParams(collective_id=0),
    )(x, Wq, K_ext, V_ext, Wo)
